# baseline (speedup 1.0000x reference)
import functools
import jax
import jax.numpy as jnp
from jax import lax
from jax.experimental import pallas as pl
from jax.experimental.pallas import tpu as pltpu
from jax.experimental.pallas import tpu_sc as plsc

B = 16384
NC = 2
NS = 16
L = 16
NW = NC * NS
BPW = B // NW


@functools.partial(
    pl.kernel,
    out_type=jax.ShapeDtypeStruct((B,), jnp.float32),
    mesh=plsc.VectorSubcoreMesh(core_axis_name="c", subcore_axis_name="s"),
    scratch_types=[
        pltpu.VMEM((BPW,), jnp.int32),
        pltpu.VMEM((BPW,), jnp.int32),
        pltpu.VMEM((BPW,), jnp.float32),
        pltpu.VMEM((BPW,), jnp.float32),
        pltpu.VMEM((BPW,), jnp.float32),
        pltpu.VMEM((3, L), jnp.float32),
        pltpu.SemaphoreType.DMA,
    ],
)
def _sc_link_scores(author_hbm, paper_hbm, src_hbm, dst_hbm, params_hbm,
                    out_hbm, sidx_v, didx_v, srow_v, drow_v, out_v, par_v,
                    sem):
    wid = lax.axis_index("s") * NC + lax.axis_index("c")
    base = wid * BPW
    pltpu.sync_copy(src_hbm.at[pl.ds(base, BPW)], sidx_v)
    pltpu.sync_copy(dst_hbm.at[pl.ds(base, BPW)], didx_v)
    pltpu.sync_copy(params_hbm, par_v)
    cp_s = pltpu.async_copy(author_hbm.at[sidx_v], srow_v, sem)
    cp_d = pltpu.async_copy(paper_hbm.at[didx_v], drow_v, sem)
    cp_s.wait()
    cp_d.wait()
    w0 = par_v[0, :]
    w1 = par_v[1, :]
    bb = par_v[2, :]
    for i in range(BPW // L):
        sl = pl.ds(i * L, L)
        out_v[sl] = srow_v[sl] * w0 + drow_v[sl] * w1 + bb
    pltpu.sync_copy(out_v, out_hbm.at[pl.ds(base, BPW)])


def kernel(author_x, paper_x, src_index, dst_index, W, b):
    params = jnp.stack([
        jnp.broadcast_to(W[0, 0], (L,)),
        jnp.broadcast_to(W[1, 0], (L,)),
        jnp.broadcast_to(b[0], (L,)),
    ])
    author_flat = jnp.reshape(jnp.swapaxes(author_x, 0, 1), (-1,))
    paper_flat = jnp.reshape(jnp.swapaxes(paper_x, 0, 1), (-1,))
    return _sc_link_scores(author_flat, paper_flat,
                           src_index, dst_index, params)


# speedup vs baseline: 3.2234x; 3.2234x over previous
import functools
import jax
import jax.numpy as jnp
from jax import lax
from jax.experimental import pallas as pl
from jax.experimental.pallas import tpu as pltpu
from jax.experimental.pallas import tpu_sc as plsc

B = 16384
NC = 2
NS = 16
L = 16
NW = NC * NS
BPW = B // NW
TAB_ROWS = 7808
TAB_SPLIT = TAB_ROWS * 128
N_TABLE = 1000000
TAIL = N_TABLE - TAB_SPLIT
AUX = 2 * TAIL + 3 * L


@functools.partial(
    pl.kernel,
    out_type=jax.ShapeDtypeStruct((B,), jnp.float32),
    mesh=plsc.VectorSubcoreMesh(core_axis_name="c", subcore_axis_name="s"),
    compiler_params=pltpu.CompilerParams(needs_layout_passes=False),
    scratch_types=[
        pltpu.VMEM((BPW,), jnp.int32),
        pltpu.VMEM((BPW,), jnp.int32),
        pltpu.VMEM((BPW,), jnp.int32),
        pltpu.VMEM((BPW,), jnp.int32),
        pltpu.VMEM((BPW,), jnp.float32),
        pltpu.VMEM((BPW,), jnp.float32),
        pltpu.VMEM((AUX,), jnp.float32),
        pltpu.VMEM((BPW,), jnp.float32),
        pltpu.SemaphoreType.DMA,
        pltpu.SemaphoreType.DMA,
        pltpu.SemaphoreType.DMA,
    ],
)
def _sc_link_scores(tab_a, tab_p, aux_hbm, src_hbm, dst_hbm,
                    out_hbm, sidx_v, didx_v, sclamp_v, dclamp_v,
                    sval_v, dval_v, aux_v, out_v, sem0, sem1, sem2):
    wid = lax.axis_index("s") * NC + lax.axis_index("c")
    base = wid * BPW
    cp_si = pltpu.async_copy(src_hbm.at[pl.ds(base, BPW)], sidx_v, sem0)
    cp_di = pltpu.async_copy(dst_hbm.at[pl.ds(base, BPW)], didx_v, sem1)
    cp_au = pltpu.async_copy(aux_hbm, aux_v, sem2)
    n_grp = BPW // L
    lim = jnp.full((L,), TAB_SPLIT - 1, jnp.int32)

    cp_si.wait()
    for i in range(n_grp):
        sl = pl.ds(i * L, L)
        sclamp_v[sl] = jnp.minimum(sidx_v[sl], lim)
    cp_sv = pltpu.async_copy(tab_a.at[sclamp_v], sval_v, sem0)

    cp_di.wait()
    for i in range(n_grp):
        sl = pl.ds(i * L, L)
        dclamp_v[sl] = jnp.minimum(didx_v[sl], lim)
    cp_dv = pltpu.async_copy(tab_p.at[dclamp_v], dval_v, sem1)

    cp_au.wait()
    w0 = aux_v[pl.ds(2 * TAIL, L)]
    w1 = aux_v[pl.ds(2 * TAIL + L, L)]
    bb = aux_v[pl.ds(2 * TAIL + 2 * L, L)]

    cp_sv.wait()
    cp_dv.wait()
    for i in range(n_grp):
        sl = pl.ds(i * L, L)
        sidx = sidx_v[sl]
        didx = didx_v[sl]
        s_tail = plsc.load_gather(
            aux_v, [jnp.clip(sidx - TAB_SPLIT, 0, TAIL - 1)])
        d_tail = plsc.load_gather(
            aux_v, [TAIL + jnp.clip(didx - TAB_SPLIT, 0, TAIL - 1)])
        sval = jnp.where(sidx >= TAB_SPLIT, s_tail, sval_v[sl])
        dval = jnp.where(didx >= TAB_SPLIT, d_tail, dval_v[sl])
        out_v[sl] = sval * w0 + dval * w1 + bb
    pltpu.sync_copy(out_v, out_hbm.at[pl.ds(base, BPW)])


def _flat_main(table):
    t = table[:TAB_SPLIT].reshape(TAB_ROWS, 128)
    t = lax.optimization_barrier(t)
    return t.reshape(-1)


def kernel(author_x, paper_x, src_index, dst_index, W, b):
    aux = jnp.concatenate([
        author_x[TAB_SPLIT:, 0],
        paper_x[TAB_SPLIT:, 0],
        jnp.broadcast_to(W[0, 0], (L,)),
        jnp.broadcast_to(W[1, 0], (L,)),
        jnp.broadcast_to(b[0], (L,)),
    ])
    return _sc_link_scores(_flat_main(author_x), _flat_main(paper_x), aux,
                           src_index, dst_index)


# speedup vs baseline: 3.3441x; 1.0374x over previous
import functools
import jax
import jax.numpy as jnp
from jax import lax
from jax.experimental import pallas as pl
from jax.experimental.pallas import tpu as pltpu
from jax.experimental.pallas import tpu_sc as plsc

B = 16384
NC = 2
NS = 16
L = 16
NW = NC * NS
BPW = B // NW
TAB_ROWS = 7808
TAB_SPLIT = TAB_ROWS * 128
N_TABLE = 1000000
TAIL = N_TABLE - TAB_SPLIT


@functools.partial(
    pl.kernel,
    out_type=jax.ShapeDtypeStruct((B,), jnp.float32),
    mesh=plsc.VectorSubcoreMesh(core_axis_name="c", subcore_axis_name="s"),
    compiler_params=pltpu.CompilerParams(needs_layout_passes=False),
    scratch_types=[
        pltpu.VMEM((BPW,), jnp.int32),
        pltpu.VMEM((BPW,), jnp.int32),
        pltpu.VMEM((BPW,), jnp.int32),
        pltpu.VMEM((BPW,), jnp.int32),
        pltpu.VMEM((BPW,), jnp.float32),
        pltpu.VMEM((BPW,), jnp.float32),
        pltpu.VMEM((TAIL,), jnp.float32),
        pltpu.VMEM((TAIL,), jnp.float32),
        pltpu.VMEM((BPW,), jnp.float32),
        pltpu.VMEM((3, L), jnp.float32),
        pltpu.SemaphoreType.DMA,
        pltpu.SemaphoreType.DMA,
        pltpu.SemaphoreType.DMA,
    ],
)
def _sc_link_scores(tab_a, tail_a, tab_p, tail_p, src_hbm, dst_hbm,
                    params_hbm, out_hbm, sidx_v, didx_v, sclamp_v, dclamp_v,
                    sval_v, dval_v, ta_v, tp_v, out_v, par_v,
                    sem0, sem1, sem2):
    wid = lax.axis_index("s") * NC + lax.axis_index("c")
    base = wid * BPW
    cp_si = pltpu.async_copy(src_hbm.at[pl.ds(base, BPW)], sidx_v, sem0)
    cp_di = pltpu.async_copy(dst_hbm.at[pl.ds(base, BPW)], didx_v, sem1)
    cp_pr = pltpu.async_copy(params_hbm, par_v, sem2)
    cp_ta = pltpu.async_copy(tail_a, ta_v, sem2)
    cp_tp = pltpu.async_copy(tail_p, tp_v, sem2)
    n_grp = BPW // L
    lim = jnp.full((L,), TAB_SPLIT - 1, jnp.int32)

    cp_si.wait()
    for i in range(n_grp):
        sl = pl.ds(i * L, L)
        sclamp_v[sl] = jnp.minimum(sidx_v[sl], lim)
    cp_sv = pltpu.async_copy(tab_a.at[sclamp_v], sval_v, sem0)

    cp_di.wait()
    for i in range(n_grp):
        sl = pl.ds(i * L, L)
        dclamp_v[sl] = jnp.minimum(didx_v[sl], lim)
    cp_dv = pltpu.async_copy(tab_p.at[dclamp_v], dval_v, sem1)

    cp_pr.wait()
    cp_ta.wait()
    cp_tp.wait()
    w0 = par_v[0, :]
    w1 = par_v[1, :]
    bb = par_v[2, :]

    cp_sv.wait()
    cp_dv.wait()
    for i in range(n_grp):
        sl = pl.ds(i * L, L)
        sidx = sidx_v[sl]
        didx = didx_v[sl]
        s_tail = plsc.load_gather(
            ta_v, [jnp.clip(sidx - TAB_SPLIT, 0, TAIL - 1)])
        d_tail = plsc.load_gather(
            tp_v, [jnp.clip(didx - TAB_SPLIT, 0, TAIL - 1)])
        sval = jnp.where(sidx >= TAB_SPLIT, s_tail, sval_v[sl])
        dval = jnp.where(didx >= TAB_SPLIT, d_tail, dval_v[sl])
        out_v[sl] = sval * w0 + dval * w1 + bb
    pltpu.sync_copy(out_v, out_hbm.at[pl.ds(base, BPW)])


def _flat_main(table):
    t = table[:TAB_SPLIT].reshape(TAB_ROWS, 128)
    t = lax.optimization_barrier(t)
    return t.reshape(-1)


def kernel(author_x, paper_x, src_index, dst_index, W, b):
    params = jnp.broadcast_to(
        jnp.concatenate([W[:, 0], b])[:, None], (3, L))
    return _sc_link_scores(_flat_main(author_x), author_x[TAB_SPLIT:, 0],
                           _flat_main(paper_x), paper_x[TAB_SPLIT:, 0],
                           src_index, dst_index, params)
